# speedup vs baseline: 152.9416x; 152.9416x over previous
import functools

import jax
import jax.numpy as jnp
from jax import lax
from jax.experimental import pallas as pl
from jax.experimental.pallas import tpu as pltpu
from jax.experimental.pallas import tpu_sc as plsc

_K = 100000
_B = 16384
_NS = 50
_N = _B * _NS
_NW = 32
_NPW = _N // _NW
_L = 16


def _body(prob_hbm, alias_hbm, kk_hbm, u_hbm, out_hbm,
          kk_v, u_v, p_v, a_v, sem):
    c = lax.axis_index("c")
    s = lax.axis_index("s")
    wid = s * 2 + c
    base = wid * _NPW

    pltpu.sync_copy(kk_hbm.at[pl.ds(base, _NPW)], kk_v)
    pltpu.sync_copy(u_hbm.at[pl.ds(base, _NPW)], u_v)

    pltpu.async_copy(prob_hbm.at[kk_v], p_v, sem).wait()
    pltpu.async_copy(alias_hbm.at[kk_v], a_v, sem).wait()

    def step(i, carry):
        sl = pl.ds(i * _L, _L)
        b = u_v[sl] < p_v[sl]
        a_v[sl] = jnp.where(b, kk_v[sl], a_v[sl])
        return carry

    lax.fori_loop(0, _NPW // _L, step, 0)

    pltpu.sync_copy(a_v, out_hbm.at[pl.ds(base, _NPW)])


@jax.jit
def _sample(prob, alias, kk, u):
    mesh = plsc.VectorSubcoreMesh(core_axis_name="c", subcore_axis_name="s")
    f = pl.kernel(
        _body,
        mesh=mesh,
        out_type=jax.ShapeDtypeStruct((_N,), jnp.int32),
        scratch_types=[
            pltpu.VMEM((_NPW,), jnp.int32),
            pltpu.VMEM((_NPW,), jnp.float32),
            pltpu.VMEM((_NPW,), jnp.float32),
            pltpu.VMEM((_NPW,), jnp.int32),
            pltpu.SemaphoreType.DMA,
        ],
    )
    return f(prob, alias, kk, u)


def kernel(prob, alias, kk, u):
    return _sample(prob, alias, kk, u).reshape(_B, _NS)


# speedup vs baseline: 203.4337x; 1.3301x over previous
import jax
import jax.numpy as jnp
from jax import lax
from jax.experimental import pallas as pl
from jax.experimental.pallas import tpu as pltpu
from jax.experimental.pallas import tpu_sc as plsc

_K = 100000
_B = 16384
_NS = 50
_N = _B * _NS
_NW = 32
_NPW = _N // _NW
_L = 16
_NCHUNK = 8
_C = _NPW // _NCHUNK


def _body(prob_hbm, alias_hbm, kk_hbm, u_hbm, out_hbm,
          prob_t, kk0, kk1, u0, u1, a0, a1,
          sk0, sk1, su0, su1, sg0, sg1, so0, so1):
    c = lax.axis_index("c")
    s = lax.axis_index("s")
    wid = s * 2 + c
    base = wid * _NPW

    kk_v = (kk0, kk1)
    u_v = (u0, u1)
    a_v = (a0, a1)
    sk = (sk0, sk1)
    su = (su0, su1)
    sg = (sg0, sg1)
    so = (so0, so1)

    def in_copies(i):
        off = base + i * _C
        j = i % 2
        ck = pltpu.async_copy(kk_hbm.at[pl.ds(off, _C)], kk_v[j], sk[j])
        cu = pltpu.async_copy(u_hbm.at[pl.ds(off, _C)], u_v[j], su[j])
        return ck, cu

    ck, cu = in_copies(0)
    pltpu.sync_copy(prob_hbm, prob_t)
    ck.wait()
    cu.wait()
    g = pltpu.async_copy(alias_hbm.at[kk_v[0]], a_v[0], sg[0])
    ck, cu = in_copies(1)

    out_cp = [None, None]
    for i in range(_NCHUNK):
        j = i % 2
        nj = 1 - j
        if i + 1 < _NCHUNK:
            ck.wait()
            cu.wait()
            if out_cp[nj] is not None:
                out_cp[nj].wait()
            g_next = pltpu.async_copy(alias_hbm.at[kk_v[nj]], a_v[nj], sg[nj])
        g.wait()

        def step(t, carry, j=j):
            sl = pl.ds(t * _L, _L)
            kkv = kk_v[j][sl]
            p = plsc.load_gather(prob_t, [kkv])
            b = u_v[j][sl] < p
            a_v[j][sl] = jnp.where(b, kkv, a_v[j][sl])
            return carry

        lax.fori_loop(0, _C // _L, step, 0)

        out_cp[j] = pltpu.async_copy(
            a_v[j], out_hbm.at[pl.ds(base + i * _C, _C)], so[j])
        if i + 2 < _NCHUNK:
            ck, cu = in_copies(i + 2)
        if i + 1 < _NCHUNK:
            g = g_next

    out_cp[0].wait()
    out_cp[1].wait()


@jax.jit
def _sample(prob, alias, kk, u):
    mesh = plsc.VectorSubcoreMesh(core_axis_name="c", subcore_axis_name="s")
    f = pl.kernel(
        _body,
        mesh=mesh,
        compiler_params=pltpu.CompilerParams(needs_layout_passes=False),
        out_type=jax.ShapeDtypeStruct((_N,), jnp.int32),
        scratch_types=[
            pltpu.VMEM((_K,), jnp.float32),
            pltpu.VMEM((_C,), jnp.int32),
            pltpu.VMEM((_C,), jnp.int32),
            pltpu.VMEM((_C,), jnp.float32),
            pltpu.VMEM((_C,), jnp.float32),
            pltpu.VMEM((_C,), jnp.int32),
            pltpu.VMEM((_C,), jnp.int32),
        ] + [pltpu.SemaphoreType.DMA] * 8,
    )
    return f(prob, alias, kk, u)


def kernel(prob, alias, kk, u):
    return _sample(prob, alias, kk, u).reshape(_B, _NS)


# speedup vs baseline: 270.9239x; 1.3318x over previous
import jax
import jax.numpy as jnp
from jax import lax
from jax.experimental import pallas as pl
from jax.experimental.pallas import tpu as pltpu
from jax.experimental.pallas import tpu_sc as plsc

_K = 100000
_B = 16384
_NS = 50
_N = _B * _NS
_NW = 32
_NPW = _N // _NW
_L = 16
_NCHUNK = 8
_C = _NPW // _NCHUNK
_ST = 6400


def _body(prob_hbm, alias_hbm, kk_hbm, u_hbm, out_hbm,
          prob_t, alias_s, kk0, kk1, u0, u1, a0, a1,
          sk0, sk1, su0, su1, sg0, sg1, so0, so1):
    c = lax.axis_index("c")
    s = lax.axis_index("s")
    wid = s * 2 + c
    base = wid * _NPW

    kk_v = (kk0, kk1)
    u_v = (u0, u1)
    a_v = (a0, a1)
    sk = (sk0, sk1)
    su = (su0, su1)
    sg = (sg0, sg1)
    so = (so0, so1)

    def in_copies(i):
        off = base + i * _C
        j = i % 2
        ck = pltpu.async_copy(kk_hbm.at[pl.ds(off, _C)], kk_v[j], sk[j])
        cu = pltpu.async_copy(u_hbm.at[pl.ds(off, _C)], u_v[j], su[j])
        return ck, cu

    ck, cu = in_copies(0)
    with jax.named_scope("stage_tables"):
        o0 = jnp.where(s < 15, s * _ST, _K - 4000)
        o1 = jnp.where(s < 15, s * _ST + _C, _K - _C)
        pltpu.sync_copy(alias_hbm.at[pl.ds(o0, _C)], a0)
        pltpu.sync_copy(alias_hbm.at[pl.ds(o1, _C)], a1)
        pltpu.sync_copy(a0, alias_s.at[pl.ds(o0, _C)])
        pltpu.sync_copy(a1, alias_s.at[pl.ds(o1, _C)])
        pltpu.sync_copy(prob_hbm, prob_t)
        plsc.subcore_barrier()
    ck.wait()
    cu.wait()
    g = pltpu.async_copy(alias_s.at[kk_v[0]], a_v[0], sg[0])
    ck, cu = in_copies(1)

    out_cp = [None, None]
    for i in range(_NCHUNK):
        j = i % 2
        nj = 1 - j
        with jax.named_scope(f"hdr{i}"):
            if i + 1 < _NCHUNK:
                ck.wait()
                cu.wait()
                if out_cp[nj] is not None:
                    out_cp[nj].wait()
                g_next = pltpu.async_copy(alias_s.at[kk_v[nj]], a_v[nj],
                                          sg[nj])
            g.wait()

        with jax.named_scope(f"sel{i}"):
            def step(t, carry, j=j):
                sl = pl.ds(t * _L, _L)
                kkv = kk_v[j][sl]
                p = plsc.load_gather(prob_t, [kkv])
                b = u_v[j][sl] < p
                a_v[j][sl] = jnp.where(b, kkv, a_v[j][sl])
                return carry

            lax.fori_loop(0, _C // _L, step, 0)

        out_cp[j] = pltpu.async_copy(
            a_v[j], out_hbm.at[pl.ds(base + i * _C, _C)], so[j])
        if i + 2 < _NCHUNK:
            ck, cu = in_copies(i + 2)
        if i + 1 < _NCHUNK:
            g = g_next

    out_cp[0].wait()
    out_cp[1].wait()


@jax.jit
def _sample(prob, alias, kk, u):
    mesh = plsc.VectorSubcoreMesh(core_axis_name="c", subcore_axis_name="s")
    f = pl.kernel(
        _body,
        mesh=mesh,
        compiler_params=pltpu.CompilerParams(needs_layout_passes=False),
        out_type=jax.ShapeDtypeStruct((_N,), jnp.int32),
        scratch_types=[
            pltpu.VMEM((_K,), jnp.float32),
            pltpu.VMEM_SHARED((_K,), jnp.int32),
            pltpu.VMEM((_C,), jnp.int32),
            pltpu.VMEM((_C,), jnp.int32),
            pltpu.VMEM((_C,), jnp.float32),
            pltpu.VMEM((_C,), jnp.float32),
            pltpu.VMEM((_C,), jnp.int32),
            pltpu.VMEM((_C,), jnp.int32),
        ] + [pltpu.SemaphoreType.DMA] * 8,
    )
    return f(prob, alias, kk, u)


def kernel(prob, alias, kk, u):
    return _sample(prob, alias, kk, u).reshape(_B, _NS)


# speedup vs baseline: 275.2653x; 1.0160x over previous
import jax
import jax.numpy as jnp
from jax import lax
from jax.experimental import pallas as pl
from jax.experimental.pallas import tpu as pltpu
from jax.experimental.pallas import tpu_sc as plsc

_K = 100000
_B = 16384
_NS = 50
_N = _B * _NS
_NW = 32
_NPW = _N // _NW
_L = 16
_NCHUNK = 16
_C = _NPW // _NCHUNK
_RC = _C // _NS
_ST = 6400


def _body(prob_hbm, alias_hbm, kk_hbm, u_hbm, out_hbm,
          prob_t, alias_s, kk0, kk1, u0, u1, a0, a1, o0, o1,
          sk0, sk1, su0, su1, sg0, sg1, so0, so1):
    c = lax.axis_index("c")
    s = lax.axis_index("s")
    wid = s * 2 + c
    base = wid * _NPW
    rbase = wid * (_NPW // _NS)

    kk_v = (kk0, kk1)
    u_v = (u0, u1)
    a_v = (a0, a1)
    o_v = (o0, o1)
    sk = (sk0, sk1)
    su = (su0, su1)
    sg = (sg0, sg1)
    so = (so0, so1)

    def in_copies(i):
        off = base + i * _C
        j = i % 2
        ck = pltpu.async_copy(kk_hbm.at[pl.ds(off, _C)], kk_v[j], sk[j])
        cu = pltpu.async_copy(u_hbm.at[pl.ds(off, _C)], u_v[j], su[j])
        return ck, cu

    ck, cu = in_copies(0)
    with jax.named_scope("stage_tables"):
        cp = pltpu.async_copy(prob_hbm, prob_t, so0)
        o0_ = jnp.where(s < 15, s * _ST, _K - 4000)
        o1_ = jnp.where(s < 15, s * _ST + 3200, _K - 3200)
        b0 = kk1
        pltpu.sync_copy(alias_hbm.at[pl.ds(o0_, 1600)], b0)
        pltpu.sync_copy(b0, alias_s.at[pl.ds(o0_, 1600)])
        pltpu.sync_copy(alias_hbm.at[pl.ds(o0_ + 1600, 1600)], b0)
        pltpu.sync_copy(b0, alias_s.at[pl.ds(o0_ + 1600, 1600)])
        pltpu.sync_copy(alias_hbm.at[pl.ds(o1_, 1600)], b0)
        pltpu.sync_copy(b0, alias_s.at[pl.ds(o1_, 1600)])
        pltpu.sync_copy(alias_hbm.at[pl.ds(o1_ + 1600, 1600)], b0)
        pltpu.sync_copy(b0, alias_s.at[pl.ds(o1_ + 1600, 1600)])
        cp.wait()
        plsc.subcore_barrier()
    ck.wait()
    cu.wait()
    g = pltpu.async_copy(alias_s.at[kk_v[0]], a_v[0], sg[0])
    ck, cu = in_copies(1)

    out_cp = [None, None]
    for i in range(_NCHUNK):
        j = i % 2
        nj = 1 - j
        with jax.named_scope(f"hdr{i}"):
            if i + 1 < _NCHUNK:
                ck.wait()
                cu.wait()
                g_next = pltpu.async_copy(alias_s.at[kk_v[nj]], a_v[nj],
                                          sg[nj])
            if out_cp[j] is not None:
                out_cp[j].wait()
            g.wait()

        with jax.named_scope(f"sel{i}"):
            def step(t, rc, j=j):
                row, col = rc
                sl = pl.ds(t * _L, _L)
                kkv = kk_v[j][sl]
                p = plsc.load_gather(prob_t, [kkv])
                b = u_v[j][sl] < p
                val = jnp.where(b, kkv, a_v[j][sl])
                plsc.store_scatter(o_v[j], [row, col], val)
                col = col + _L
                wrap = col >= _NS
                col = jnp.where(wrap, col - _NS, col)
                row = row + wrap.astype(jnp.int32)
                return row, col

            lax.fori_loop(0, _C // _L, step,
                          (jnp.zeros((_L,), jnp.int32),
                           lax.iota(jnp.int32, _L)))

        out_cp[j] = pltpu.async_copy(
            o_v[j], out_hbm.at[pl.ds(rbase + i * _RC, _RC), :], so[j])
        if i + 2 < _NCHUNK:
            ck, cu = in_copies(i + 2)
        if i + 1 < _NCHUNK:
            g = g_next

    out_cp[0].wait()
    out_cp[1].wait()


@jax.jit
def _sample(prob, alias, kk, u):
    mesh = plsc.VectorSubcoreMesh(core_axis_name="c", subcore_axis_name="s")
    f = pl.kernel(
        _body,
        mesh=mesh,
        compiler_params=pltpu.CompilerParams(needs_layout_passes=False),
        out_type=jax.ShapeDtypeStruct((_B, 128), jnp.int32),
        scratch_types=[
            pltpu.VMEM((_K,), jnp.float32),
            pltpu.VMEM_SHARED((_K,), jnp.int32),
            pltpu.VMEM((_C,), jnp.int32),
            pltpu.VMEM((_C,), jnp.int32),
            pltpu.VMEM((_C,), jnp.float32),
            pltpu.VMEM((_C,), jnp.float32),
            pltpu.VMEM((_C,), jnp.int32),
            pltpu.VMEM((_C,), jnp.int32),
            pltpu.VMEM((_RC, 128), jnp.int32),
            pltpu.VMEM((_RC, 128), jnp.int32),
        ] + [pltpu.SemaphoreType.DMA] * 8,
    )
    return f(prob, alias, kk, u)


def kernel(prob, alias, kk, u):
    return _sample(prob, alias, kk, u)[:, :_NS]
